# baseline (speedup 1.0000x reference)
import functools

import jax
import jax.numpy as jnp
from jax import lax
from jax.experimental import pallas as pl
from jax.experimental.pallas import tpu as pltpu
from jax.experimental.pallas import tpu_sc as plsc

NUM_USERS = 1000000
LATENT_DIM = 16
BATCH = 16384
NC = 2
NS = 16
NW = NC * NS
BPW = BATCH // NW
GCHUNK = 128
NG = BPW // GCHUNK


@functools.partial(
    pl.kernel,
    mesh=plsc.VectorSubcoreMesh(core_axis_name="c", subcore_axis_name="s"),
    compiler_params=pltpu.CompilerParams(
        needs_layout_passes=False, use_tc_tiling_on_sc=False),
    out_type=jax.ShapeDtypeStruct((BATCH,), jnp.float32),
    scratch_types=[
        pltpu.VMEM((NG, GCHUNK), jnp.int32),
        pltpu.VMEM((NG, GCHUNK), jnp.int32),
        pltpu.VMEM((BPW, LATENT_DIM), jnp.float32),
        pltpu.VMEM((BPW, LATENT_DIM), jnp.float32),
        pltpu.VMEM((BPW,), jnp.float32),
        pltpu.VMEM((BPW,), jnp.float32),
        pltpu.VMEM((BPW,), jnp.float32),
        pltpu.VMEM((16 * 16,), jnp.float32),
        pltpu.SemaphoreType.DMA,
    ],
)
def _fm_sc(uids_hbm, iids_hbm, table_hbm, lin_hbm, out_hbm,
           uidx_v, iidx_v, urows_v, irows_v, lwu_v, lwi_v, out_v, p_v, sem):
    wid = lax.axis_index("s") * NC + lax.axis_index("c")
    base = pl.multiple_of(wid * BPW, BPW)
    rowblk = pl.multiple_of(wid * NG, NG)

    pltpu.sync_copy(uids_hbm.at[pl.ds(rowblk, NG)], uidx_v)
    pltpu.sync_copy(iids_hbm.at[pl.ds(rowblk, NG)], iidx_v)

    for j in range(NG):
        for k in range(GCHUNK // 16):
            sl = (j, pl.ds(k * 16, 16))
            iidx_v[sl] = iidx_v[sl] + NUM_USERS

    copies = []
    for j in range(NG):
        dst = pl.ds(j * GCHUNK, GCHUNK)
        copies.append(pltpu.async_copy(
            table_hbm.at[uidx_v.at[j]], urows_v.at[dst], sem))
        copies.append(pltpu.async_copy(
            table_hbm.at[iidx_v.at[j]], irows_v.at[dst], sem))
        copies.append(pltpu.async_copy(
            lin_hbm.at[uidx_v.at[j]], lwu_v.at[dst], sem))
        copies.append(pltpu.async_copy(
            lin_hbm.at[iidx_v.at[j]], lwi_v.at[dst], sem))
    for c in copies:
        c.wait()

    iota16 = lax.iota(jnp.int32, 16)

    def body(g, carry):
        off = pl.multiple_of(g * 16, 16)
        for j in range(16):
            u = urows_v[off + j, :]
            v = irows_v[off + j, :]
            p_v[pl.ds(j * 16, 16)] = u * v
        acc = lwu_v[pl.ds(off, 16)] + lwi_v[pl.ds(off, 16)]
        for d in range(LATENT_DIM):
            acc = acc + plsc.load_gather(p_v, [iota16 * 16 + d])
        out_v[pl.ds(off, 16)] = acc
        return carry

    lax.fori_loop(0, BPW // 16, body, 0)

    pltpu.sync_copy(out_v, out_hbm.at[pl.ds(base, BPW)])


def kernel(uids, iids, table, linear_w, bias):
    uids2 = uids.astype(jnp.int32).reshape(BATCH // GCHUNK, GCHUNK)
    iids2 = iids.astype(jnp.int32).reshape(BATCH // GCHUNK, GCHUNK)
    lin = linear_w.reshape(-1)
    out = _fm_sc(uids2, iids2, table, lin)
    return out + bias[0]


# speedup vs baseline: 7.4955x; 7.4955x over previous
import functools

import jax
import jax.numpy as jnp
from jax import lax
from jax.experimental import pallas as pl
from jax.experimental.pallas import tpu as pltpu
from jax.experimental.pallas import tpu_sc as plsc

NUM_USERS = 1000000
NUM_ROWS = 2000000
LATENT_DIM = 16
BATCH = 16384
NC = 2
NS = 16
NW = NC * NS
BPW = BATCH // NW
GCHUNK = 128
NG = BPW // GCHUNK
SLAB = NUM_ROWS * 8
FLAT_N = NUM_ROWS * LATENT_DIM


@functools.partial(
    pl.kernel,
    mesh=plsc.VectorSubcoreMesh(core_axis_name="c", subcore_axis_name="s"),
    compiler_params=pltpu.CompilerParams(
        needs_layout_passes=False, use_tc_tiling_on_sc=False),
    out_type=jax.ShapeDtypeStruct((BATCH,), jnp.float32),
    scratch_types=[
        pltpu.VMEM((BPW,), jnp.int32),
        pltpu.VMEM((BPW,), jnp.int32),
        pltpu.VMEM((BPW,), jnp.int32),
        pltpu.VMEM((BPW,), jnp.int32),
        pltpu.VMEM((LATENT_DIM, BPW), jnp.float32),
        pltpu.VMEM((LATENT_DIM, BPW), jnp.float32),
        pltpu.VMEM((BPW,), jnp.float32),
        pltpu.VMEM((BPW,), jnp.float32),
        pltpu.VMEM((BPW,), jnp.float32),
        pltpu.SemaphoreType.DMA,
    ],
)
def _fm_sc(uids_hbm, iids_hbm, flat_hbm, lin_hbm, out_hbm,
           uidx_v, iidx_v, uoff_v, ioff_v, ucols_v, icols_v,
           lwu_v, lwi_v, out_v, sem):
    wid = lax.axis_index("s") * NC + lax.axis_index("c")
    base = pl.multiple_of(wid * BPW, BPW)

    pltpu.sync_copy(uids_hbm.at[pl.ds(base, BPW)], uidx_v)
    pltpu.sync_copy(iids_hbm.at[pl.ds(base, BPW)], iidx_v)

    for k in range(BPW // 16):
        sl = pl.ds(k * 16, 16)
        u = uidx_v[sl]
        uoff_v[sl] = ((u >> 7) << 10) + (u & 127)
        i = iidx_v[sl] + NUM_USERS
        iidx_v[sl] = i
        ioff_v[sl] = ((i >> 7) << 10) + (i & 127)

    copies = []
    for j in range(NG):
        jsl = pl.ds(j * GCHUNK, GCHUNK)
        copies.append(pltpu.async_copy(
            lin_hbm.at[uidx_v.at[jsl]], lwu_v.at[jsl], sem))
        copies.append(pltpu.async_copy(
            lin_hbm.at[iidx_v.at[jsl]], lwi_v.at[jsl], sem))
    for d in range(LATENT_DIM):
        base_d = (d >> 3) * SLAB + (d & 7) * 128
        src = flat_hbm.at[pl.ds(base_d, FLAT_N - base_d)]
        for j in range(NG):
            jsl = pl.ds(j * GCHUNK, GCHUNK)
            copies.append(pltpu.async_copy(
                src.at[uoff_v.at[jsl]], ucols_v.at[d, jsl], sem))
            copies.append(pltpu.async_copy(
                src.at[ioff_v.at[jsl]], icols_v.at[d, jsl], sem))
    for c in copies:
        c.wait()

    def body(g, carry):
        off = pl.multiple_of(g * 16, 16)
        sl = pl.ds(off, 16)
        acc = lwu_v[sl] + lwi_v[sl]
        for d in range(LATENT_DIM):
            acc = acc + ucols_v[d, sl] * icols_v[d, sl]
        out_v[sl] = acc
        return carry

    lax.fori_loop(0, BPW // 16, body, 0)

    pltpu.sync_copy(out_v, out_hbm.at[pl.ds(base, BPW)])


def kernel(uids, iids, table, linear_w, bias):
    flat = (table.T.reshape(2, 8, NUM_ROWS // 128, 128)
            .transpose(0, 2, 1, 3).reshape(-1))
    lin = linear_w.T.reshape(-1)
    out = _fm_sc(uids.astype(jnp.int32), iids.astype(jnp.int32), flat, lin)
    return out + bias[0]


# speedup vs baseline: 19.9474x; 2.6612x over previous
import functools

import jax
import jax.numpy as jnp
from jax import lax
from jax.experimental import pallas as pl
from jax.experimental.pallas import tpu as pltpu
from jax.experimental.pallas import tpu_sc as plsc

NUM_USERS = 1000000
NUM_ROWS = 2000000
LATENT_DIM = 16
BATCH = 16384
NC = 2
NS = 16
NW = NC * NS
BPW = BATCH // NW
GCHUNK = 128
NG = BPW // GCHUNK
SLAB = NUM_ROWS * 8
FLAT_N = NUM_ROWS * LATENT_DIM


@functools.partial(
    pl.kernel,
    mesh=plsc.VectorSubcoreMesh(core_axis_name="c", subcore_axis_name="s"),
    compiler_params=pltpu.CompilerParams(
        needs_layout_passes=False, use_tc_tiling_on_sc=True),
    out_type=jax.ShapeDtypeStruct((BATCH,), jnp.float32),
    scratch_types=[
        pltpu.VMEM((BPW,), jnp.int32),
        pltpu.VMEM((BPW,), jnp.int32),
        pltpu.VMEM((BPW,), jnp.int32),
        pltpu.VMEM((BPW,), jnp.int32),
        pltpu.VMEM((LATENT_DIM, BPW), jnp.float32),
        pltpu.VMEM((LATENT_DIM, BPW), jnp.float32),
        pltpu.VMEM((BPW,), jnp.float32),
        pltpu.VMEM((BPW,), jnp.float32),
        pltpu.VMEM((BPW,), jnp.float32),
        pltpu.SemaphoreType.DMA,
    ],
)
def _fm_sc(uids_hbm, iids_hbm, flat_hbm, lin_hbm, out_hbm,
           uidx_v, iidx_v, uoff_v, ioff_v, ucols_v, icols_v,
           lwu_v, lwi_v, out_v, sem):
    wid = lax.axis_index("s") * NC + lax.axis_index("c")
    base = pl.multiple_of(wid * BPW, BPW)

    pltpu.sync_copy(uids_hbm.at[pl.ds(base, BPW)], uidx_v)
    pltpu.sync_copy(iids_hbm.at[pl.ds(base, BPW)], iidx_v)

    for k in range(BPW // 16):
        sl = pl.ds(k * 16, 16)
        u = uidx_v[sl]
        uoff_v[sl] = ((u >> 7) << 10) + (u & 127)
        i = iidx_v[sl] + NUM_USERS
        iidx_v[sl] = i
        ioff_v[sl] = ((i >> 7) << 10) + (i & 127)

    copies = []
    lin_row = lin_hbm.at[0]
    for j in range(NG):
        jsl = pl.ds(j * GCHUNK, GCHUNK)
        copies.append(pltpu.async_copy(
            lin_row.at[uidx_v.at[jsl]], lwu_v.at[jsl], sem))
        copies.append(pltpu.async_copy(
            lin_row.at[iidx_v.at[jsl]], lwi_v.at[jsl], sem))
    for d in range(LATENT_DIM):
        base_d = (d >> 3) * SLAB + (d & 7) * 128
        src = flat_hbm.at[pl.ds(base_d, FLAT_N - base_d)]
        for j in range(NG):
            jsl = pl.ds(j * GCHUNK, GCHUNK)
            copies.append(pltpu.async_copy(
                src.at[uoff_v.at[jsl]], ucols_v.at[d, jsl], sem))
            copies.append(pltpu.async_copy(
                src.at[ioff_v.at[jsl]], icols_v.at[d, jsl], sem))
    for c in copies:
        c.wait()

    def body(g, carry):
        off = pl.multiple_of(g * 16, 16)
        sl = pl.ds(off, 16)
        acc = lwu_v[sl] + lwi_v[sl]
        for d in range(LATENT_DIM):
            acc = acc + ucols_v[d, sl] * icols_v[d, sl]
        out_v[sl] = acc
        return carry

    lax.fori_loop(0, BPW // 16, body, 0)

    pltpu.sync_copy(out_v, out_hbm.at[pl.ds(base, BPW)])


def kernel(uids, iids, table, linear_w, bias):
    flat = (table.T.reshape(2, 8, NUM_ROWS // 128, 128)
            .transpose(0, 2, 1, 3).reshape(-1))
    out = _fm_sc(uids.astype(jnp.int32), iids.astype(jnp.int32), flat,
                 linear_w.T)
    return out + bias[0]
